# Initial kernel scaffold; baseline (speedup 1.0000x reference)
import jax
import jax.numpy as jnp
from jax.experimental import pallas as pl


def kernel(x, edge_index, batch, W1, b1, W2, b2, W3, b3, Wc1, bc1, Wc2, bc2):
    raise NotImplementedError("write your pallas kernel here")


# speedup vs baseline: 20.9155x; 20.9155x over previous
import functools
import jax
import jax.numpy as jnp
from jax import lax
from jax.experimental import pallas as pl
from jax.experimental.pallas import tpu as pltpu
from jax.experimental.pallas import tpu_sc as plsc

_N = 10000
_E = 320000
_FIN = 128
_H = 64
_C = 10
_G = 64

_NC = 2
_NS = 16
_NW = _NC * _NS
_EPT = _E // _NW
_B = 80
_NCH = _EPT // _B
_RPT = _N // _NS
_DEGW = 16

_mesh = plsc.VectorSubcoreMesh(core_axis_name="c", subcore_axis_name="s")


@functools.partial(
    pl.kernel,
    out_type=jax.ShapeDtypeStruct((_NC, _NS, _RPT, _DEGW), jnp.float32),
    mesh=_mesh,
    scratch_types=[
        pltpu.VMEM((_NCH, _B), jnp.int32),
        pltpu.VMEM((_B, _DEGW), jnp.float32),
        pltpu.VMEM_SHARED((_N, _DEGW), jnp.float32),
    ],
    compiler_params=pltpu.CompilerParams(use_tc_tiling_on_sc=False),
)
def _deg_kernel(dst_hbm, zeros_hbm, ones_hbm, out_hbm, dst_v, ones_v, acc_sh):
    c = lax.axis_index("c")
    s = lax.axis_index("s")
    pltpu.sync_copy(zeros_hbm, acc_sh.at[pl.ds(s * _RPT, _RPT)])
    pltpu.sync_copy(ones_hbm, ones_v)
    pltpu.sync_copy(dst_hbm.at[c, s], dst_v)
    plsc.subcore_barrier()

    def body(j, carry):
        pltpu.sync_copy(ones_v, acc_sh.at[dst_v.at[j]], add=True)
        return carry

    lax.fori_loop(0, _NCH, body, 0)
    plsc.subcore_barrier()
    pltpu.sync_copy(acc_sh.at[pl.ds(s * _RPT, _RPT)], out_hbm.at[c, s])


@functools.partial(
    pl.kernel,
    out_type=jax.ShapeDtypeStruct((_NC, _NS, _RPT, _H), jnp.float32),
    mesh=_mesh,
    scratch_types=[
        pltpu.VMEM((_NCH, _B), jnp.int32),
        pltpu.VMEM((_NCH, _B), jnp.int32),
        pltpu.VMEM((_B, _H), jnp.float32),
        pltpu.VMEM_SHARED((_N, _H), jnp.float32),
        pltpu.SemaphoreType.DMA,
    ],
    compiler_params=pltpu.CompilerParams(use_tc_tiling_on_sc=False),
)
def _scatter_kernel(g_hbm, src_hbm, dst_hbm, zeros_hbm, out_hbm,
                    src_v, dst_v, rows_v, acc_sh, sem):
    c = lax.axis_index("c")
    s = lax.axis_index("s")
    pltpu.sync_copy(zeros_hbm, acc_sh.at[pl.ds(s * _RPT, _RPT)])
    pltpu.sync_copy(src_hbm.at[c, s], src_v)
    pltpu.sync_copy(dst_hbm.at[c, s], dst_v)
    plsc.subcore_barrier()

    def body(j, carry):
        pltpu.async_copy(g_hbm.at[src_v.at[j]], rows_v, sem).wait()
        pltpu.sync_copy(rows_v, acc_sh.at[dst_v.at[j]], add=True)
        return carry

    lax.fori_loop(0, _NCH, body, 0)
    plsc.subcore_barrier()
    pltpu.sync_copy(acc_sh.at[pl.ds(s * _RPT, _RPT)], out_hbm.at[c, s])


def _prep_body(x_ref, w1_ref, degp_ref, g_ref, dinv_ref):
    deg = degp_ref[0, :, 0:1] + degp_ref[1, :, 0:1] + 1.0
    dinv = lax.rsqrt(deg)
    g = jnp.dot(x_ref[...], w1_ref[...],
                preferred_element_type=jnp.float32) * dinv
    g_ref[...] = g
    dinv_ref[...] = dinv


def _mid_body(accp_ref, g_ref, dinv_ref, b_ref, w_ref, gout_ref):
    dinv = dinv_ref[...]
    acc = accp_ref[0] + accp_ref[1] + g_ref[...]
    h = jnp.maximum(acc * dinv + b_ref[...], 0.0)
    gout_ref[...] = jnp.dot(h, w_ref[...],
                            preferred_element_type=jnp.float32) * dinv


def _final_body(accp_ref, g_ref, dinv_ref, b_ref, batch_ref,
                wc1_ref, bc1_ref, wc2_ref, bc2_ref, out_ref):
    dinv = dinv_ref[...]
    acc = accp_ref[0] + accp_ref[1] + g_ref[...]
    h = jnp.maximum(acc * dinv + b_ref[...], 0.0)
    seg = batch_ref[...]
    gids = lax.broadcasted_iota(jnp.int32, (_G, _N), 0)
    mask = (jnp.broadcast_to(seg, (_G, _N)) == gids).astype(jnp.float32)
    sums = jnp.dot(mask, h, preferred_element_type=jnp.float32)
    cnt = jnp.sum(mask, axis=1, keepdims=True)
    pooled = sums / jnp.maximum(cnt, 1.0)
    z = jnp.maximum(jnp.dot(pooled, wc1_ref[...],
                            preferred_element_type=jnp.float32) + bc1_ref[...],
                    0.0)
    out_ref[...] = jnp.dot(z, wc2_ref[...],
                           preferred_element_type=jnp.float32) + bc2_ref[...]


_prep_call = pl.pallas_call(
    _prep_body,
    out_shape=(jax.ShapeDtypeStruct((_N, _H), jnp.float32),
               jax.ShapeDtypeStruct((_N, 1), jnp.float32)),
)

_mid_call = pl.pallas_call(
    _mid_body,
    out_shape=jax.ShapeDtypeStruct((_N, _H), jnp.float32),
)

_final_call = pl.pallas_call(
    _final_body,
    out_shape=jax.ShapeDtypeStruct((_G, _C), jnp.float32),
)


@jax.jit
def kernel(x, edge_index, batch, W1, b1, W2, b2, W3, b3, Wc1, bc1, Wc2, bc2):
    src = edge_index[0].reshape(_NC, _NS, _NCH, _B)
    dst = edge_index[1].reshape(_NC, _NS, _NCH, _B)

    zeros_deg = jnp.zeros((_RPT, _DEGW), jnp.float32)
    ones_deg = jnp.ones((_B, _DEGW), jnp.float32)
    zeros_h = jnp.zeros((_RPT, _H), jnp.float32)

    degp = _deg_kernel(dst, zeros_deg, ones_deg).reshape(_NC, _N, _DEGW)
    g1, dinv = _prep_call(x, W1, degp)

    def scatter(g):
        return _scatter_kernel(g, src, dst, zeros_h).reshape(_NC, _N, _H)

    acc1 = scatter(g1)
    g2 = _mid_call(acc1, g1, dinv, b1.reshape(1, _H), W2)
    acc2 = scatter(g2)
    g3 = _mid_call(acc2, g2, dinv, b2.reshape(1, _H), W3)
    acc3 = scatter(g3)

    out = _final_call(acc3, g3, dinv, b3.reshape(1, _H),
                      batch.reshape(1, _N), Wc1, bc1.reshape(1, _H // 2),
                      Wc2, bc2.reshape(1, _C))
    return out
